# baseline (speedup 1.0000x reference)
import functools
import math

import jax
import jax.numpy as jnp
from jax import lax
from jax.experimental import pallas as pl
from jax.experimental.pallas import tpu as pltpu
from jax.experimental.pallas import tpu_sc as plsc

_H = 128
_NUM_GRAPHS = 16
_DEPTH = 3


def _rup(n, m=256):
    return ((n + m - 1) // m) * m


def _mm_kernel(a_ref, b_ref, o_ref, *, trans_a, trans_b, diag_one_n, bm, bn):
    @pl.when(pl.program_id(2) == 0)
    def _init():
        o_ref[...] = jnp.zeros_like(o_ref)
    a = a_ref[...]
    b = b_ref[...]
    if trans_a:
        dims = (((0,), (0,)), ((), ()))
    elif trans_b:
        dims = (((1,), (1,)), ((), ()))
    else:
        dims = (((1,), (0,)), ((), ()))
    acc = jax.lax.dot_general(a, b, dims, preferred_element_type=jnp.float32)
    o_ref[...] += acc.astype(o_ref.dtype)
    if diag_one_n is not None:
        @pl.when(pl.program_id(2) == pl.num_programs(2) - 1)
        def _diag():
            rows = pl.program_id(0) * bm + jax.lax.broadcasted_iota(
                jnp.int32, (bm, bn), 0)
            cols = pl.program_id(1) * bn + jax.lax.broadcasted_iota(
                jnp.int32, (bm, bn), 1)
            o_ref[...] = jnp.where((rows == cols) & (rows < diag_one_n),
                                   1.0, o_ref[...])


def _mm(a, b, trans_a=False, trans_b=False, bm=256, bn=256, bk=256,
        diag_one_n=None, out_dtype=jnp.float32):
    if trans_a:
        k, m = a.shape
    else:
        m, k = a.shape
    n = b.shape[0] if trans_b else b.shape[1]
    bm = min(bm, m)
    bn = min(bn, n)
    bk = min(bk, k)
    if trans_a:
        a_spec = pl.BlockSpec((bk, bm), lambda i, j, q: (q, i))
    else:
        a_spec = pl.BlockSpec((bm, bk), lambda i, j, q: (i, q))
    if trans_b:
        b_spec = pl.BlockSpec((bn, bk), lambda i, j, q: (j, q))
    else:
        b_spec = pl.BlockSpec((bk, bn), lambda i, j, q: (q, j))
    return pl.pallas_call(
        functools.partial(_mm_kernel, trans_a=trans_a, trans_b=trans_b,
                          diag_one_n=diag_one_n, bm=bm, bn=bn),
        grid=(m // bm, n // bn, k // bk),
        in_specs=[a_spec, b_spec],
        out_specs=pl.BlockSpec((bm, bn), lambda i, j, q: (i, j)),
        out_shape=jax.ShapeDtypeStruct((m, n), out_dtype),
        compiler_params=pltpu.CompilerParams(
            dimension_semantics=("parallel", "parallel", "arbitrary")),
    )(a, b)


def _cast_kernel(a_ref, o_ref):
    o_ref[...] = a_ref[...].astype(o_ref.dtype)


def _to_bf16(a, blk=512):
    m, n = a.shape
    bm, bn = min(blk, m), min(blk, n)
    return pl.pallas_call(
        _cast_kernel,
        grid=(m // bm, n // bn),
        in_specs=[pl.BlockSpec((bm, bn), lambda i, j: (i, j))],
        out_specs=pl.BlockSpec((bm, bn), lambda i, j: (i, j)),
        out_shape=jax.ShapeDtypeStruct((m, n), jnp.bfloat16),
    )(a)


_NW = 32
_EC = 128


def _sc_agg_body(y_hbm, src_hbm, dst_hbm, zrows_hbm, out_hbm,
                 idx_s, idx_d, rows_v, acc, sem):
    c = lax.axis_index("c")
    s = lax.axis_index("s")
    wid = s * 2 + c
    nchunks = src_hbm.shape[1]
    p0 = acc.shape[0]
    stripe = p0 // 16
    pltpu.sync_copy(zrows_hbm, acc.at[pl.ds(s * stripe, stripe)])
    plsc.subcore_barrier()

    def chunk(i, carry):
        pltpu.sync_copy(src_hbm.at[wid, i], idx_s)
        pltpu.async_copy(y_hbm.at[idx_s], rows_v, sem).wait()
        pltpu.sync_copy(dst_hbm.at[wid, i], idx_d)
        pltpu.sync_copy(rows_v, acc.at[idx_d], add=True)
        return carry

    lax.fori_loop(0, nchunks, chunk, 0)
    plsc.subcore_barrier()
    pltpu.sync_copy(acc.at[pl.ds(s * stripe, stripe)],
                    out_hbm.at[c].at[pl.ds(s * stripe, stripe)])


def _sc_edge_agg(y, srcp, dstp, zrows):
    p0 = y.shape[0]
    kern = pl.kernel(
        _sc_agg_body,
        out_type=jax.ShapeDtypeStruct((2, p0, _H), jnp.float32),
        mesh=plsc.VectorSubcoreMesh(core_axis_name="c", subcore_axis_name="s"),
        scratch_types=[
            pltpu.VMEM((_EC,), jnp.int32),
            pltpu.VMEM((_EC,), jnp.int32),
            pltpu.VMEM((_EC, _H), jnp.float32),
            pltpu.VMEM_SHARED((p0, _H), jnp.float32),
            pltpu.SemaphoreType.DMA,
        ],
    )
    return kern(y, srcp, dstp, zrows)


def _head_kernel(g_ref, lw_ref, lb_ref, bg_ref, bb_ref, ow_ref, ob_ref,
                 o_ref):
    inv = 1.0 / jnp.sqrt(1.0 + 1e-5)
    g = g_ref[...]
    for i in range(3):
        g = g * inv * bg_ref[i][None, :] + bb_ref[i][None, :]
        g = jnp.tanh(jnp.dot(g, lw_ref[i], preferred_element_type=jnp.float32)
                     + lb_ref[i][None, :])
    g = g * inv * bg_ref[3][None, :] + bb_ref[3][None, :]
    o_ref[...] = (jnp.dot(g, ow_ref[...], preferred_element_type=jnp.float32)
                  + ob_ref[0][None, :])


def _head(g0, lin_W, lin_b, bn_g, bn_b, out_W_pad, out_b_pad):
    return pl.pallas_call(
        _head_kernel,
        out_shape=jax.ShapeDtypeStruct((_NUM_GRAPHS, _H), jnp.float32),
    )(g0, lin_W, lin_b, bn_g, bn_b, out_W_pad, out_b_pad)


def _pad_rows(x, p):
    return jnp.pad(x, ((0, p - x.shape[0]), (0, 0)))


def _dense_dis(B, mask):
    deg = jnp.sum(B, axis=1) + mask
    return jnp.where(deg > 0.0, deg ** -0.5, 0.0)


def _gcn_dense(B, dis, mask, h_in, W, b, relu):
    y = dis[:, None] * _mm(h_in, W, bm=256, bn=128, bk=128)
    agg = _mm(B, y, bm=256, bn=128, bk=256) + y
    h = dis[:, None] * agg + b[None, :] * mask[:, None]
    return jnp.maximum(h, 0.0) if relu else h


def _score(h, w):
    return jnp.tanh((h @ w) / jnp.linalg.norm(w))


def kernel(x, edge_index, batch, down_W, down_b, pool_w, up_W, up_b,
           lin_W, lin_b, bn_g, bn_b, out_W, out_b):
    f32 = jnp.float32
    n0 = x.shape[0]
    L = [n0]
    for _ in range(_DEPTH):
        L.append(int(math.ceil(0.5 * L[-1])))
    P = [_rup(l) for l in L]

    src = edge_index[0]
    dst = edge_index[1]
    selfloop = src == dst

    cnt = jnp.zeros((2 * n0,), f32).at[
        dst + n0 * selfloop.astype(jnp.int32)].add(1.0)
    indeg = cnt[:n0] + cnt[n0:]
    selfc = cnt[n0:]
    dfix = jnp.where(selfc == 0.0, 2.0, 0.0)
    dis0 = (indeg + dfix) ** -0.5
    dis0p = jnp.pad(dis0, (0, P[0] - n0))

    ne = edge_index.shape[1]
    npad = _NW * _EC * int(math.ceil(ne / (_NW * _EC)))
    srcp = jnp.pad(src, (0, npad - ne)).reshape(_NW, -1, _EC).astype(jnp.int32)
    dstp = jnp.pad(dst, (0, npad - ne), constant_values=n0)
    dstp = dstp.reshape(_NW, -1, _EC).astype(jnp.int32)
    zrows = jnp.zeros((P[0] // 16, _H), f32)

    def gcn0(h_pad, W, b, relu):
        y = dis0p[:, None] * _mm(h_pad, W, bm=256, bn=128, bk=128)
        parts = _sc_edge_agg(y, srcp, dstp, zrows)
        agg = parts[0, :n0] + parts[1, :n0]
        yl = y[:n0]
        h = dis0[:, None] * (agg + dfix[:, None] * yl) + b[None, :]
        if relu:
            h = jnp.maximum(h, 0.0)
        return _pad_rows(h, P[0])

    x_pad = _pad_rows(x, P[0])
    h0 = gcn0(x_pad, down_W[0], down_b[0], relu=True)

    masks = [(jnp.arange(p) < l).astype(f32) for p, l in zip(P, L)]

    vals1, perm1 = jax.lax.top_k(_score(h0[:n0], pool_w[0]), L[1])
    inv1 = jnp.full((n0,), P[1], jnp.int32).at[perm1].set(
        jnp.arange(L[1], dtype=jnp.int32))
    keep = ~selfloop
    rd = jnp.where(keep, inv1[dst], P[1])
    rs = jnp.where(keep, inv1[src], P[1])
    ar1 = jnp.arange(L[1])
    Ar = jnp.zeros((P[1], P[0]), f32).at[rd, src].add(1.0)
    Ar = Ar.at[ar1, perm1].add(1.0)
    Ac = jnp.zeros((P[0], P[1]), f32).at[dst, rs].add(1.0)
    Ac = Ac.at[perm1, ar1].add(1.0)
    B1 = _mm(_to_bf16(Ar), _to_bf16(Ac), bm=512, bn=512, bk=1024,
             diag_one_n=L[1])
    dis1 = _dense_dis(B1, masks[1])
    h1_in = _pad_rows(h0[:n0][perm1] * vals1[:, None], P[1])
    h1 = _gcn_dense(B1, dis1, masks[1], h1_in, down_W[1], down_b[1], True)

    def next_level(B, h, lvl, pw):
        lp, lc = L[lvl - 1], L[lvl]
        pc = P[lvl]
        vals, perm = jax.lax.top_k(_score(h[:lp], pw), lc)
        S = jnp.zeros((pc, B.shape[0]), f32).at[jnp.arange(lc), perm].set(1.0)
        S16 = _to_bf16(S)
        B16 = _to_bf16(B)
        Rr = _mm(S16, B16, bm=256, bn=512, bk=256,
                 out_dtype=jnp.bfloat16)
        Rc = _mm(B16, S16, trans_b=True, bm=256, bn=256, bk=512,
                 out_dtype=jnp.bfloat16)
        Bn = _mm(Rr, Rc, bm=256, bn=256, bk=512, diag_one_n=lc)
        h_in = _pad_rows(h[:lp][perm] * vals[:, None], pc)
        return Bn, h_in, S

    B2, h2_in, S2 = next_level(B1, h1, 2, pool_w[1])
    dis2 = _dense_dis(B2, masks[2])
    h2 = _gcn_dense(B2, dis2, masks[2], h2_in, down_W[2], down_b[2], True)

    B3, h3_in, S3 = next_level(B2, h2, 3, pool_w[2])
    dis3 = _dense_dis(B3, masks[3])
    h3 = _gcn_dense(B3, dis3, masks[3], h3_in, down_W[3], down_b[3], True)

    u = _mm(S3, h3, trans_a=True, bm=256, bn=128, bk=256)
    h = _gcn_dense(B2, dis2, masks[2], h2 + u, up_W[0], up_b[0], True)

    u = _mm(S2, h, trans_a=True, bm=256, bn=128, bk=256)
    h = _gcn_dense(B1, dis1, masks[1], h1 + u, up_W[1], up_b[1], True)

    u = jnp.zeros((n0, _H), f32).at[perm1].set(h[:L[1]])
    h = gcn0(_pad_rows(h0[:n0] + u, P[0]), up_W[2], up_b[2], relu=False)

    onehot = (batch[None, :] == jnp.arange(_NUM_GRAPHS)[:, None]).astype(f32)
    onehot = jnp.pad(onehot, ((0, 0), (0, P[0] - n0)))
    g0 = _mm(onehot, h, bm=16, bn=128, bk=256)
    ow = jnp.pad(out_W, ((0, 0), (0, _H - out_W.shape[1])))
    ob = jnp.pad(out_b, (0, _H - out_b.shape[0]))[None, :]
    out = _head(g0, lin_W, lin_b, bn_g, bn_b, ow, ob)
    return out[:, :out_W.shape[1]]


# speedup vs baseline: 1.1097x; 1.1097x over previous
import functools
import math

import jax
import jax.numpy as jnp
from jax import lax
from jax.experimental import pallas as pl
from jax.experimental.pallas import tpu as pltpu
from jax.experimental.pallas import tpu_sc as plsc

_H = 128
_NUM_GRAPHS = 16
_DEPTH = 3


def _rup(n, m=256):
    return ((n + m - 1) // m) * m


def _mm_kernel(a_ref, b_ref, o_ref, *, trans_a, trans_b, diag_one_n, bm, bn):
    @pl.when(pl.program_id(2) == 0)
    def _init():
        o_ref[...] = jnp.zeros_like(o_ref)
    a = a_ref[...]
    b = b_ref[...]
    if trans_a:
        dims = (((0,), (0,)), ((), ()))
    elif trans_b:
        dims = (((1,), (1,)), ((), ()))
    else:
        dims = (((1,), (0,)), ((), ()))
    acc = jax.lax.dot_general(a, b, dims, preferred_element_type=jnp.float32)
    o_ref[...] += acc.astype(o_ref.dtype)
    if diag_one_n is not None:
        @pl.when(pl.program_id(2) == pl.num_programs(2) - 1)
        def _diag():
            rows = pl.program_id(0) * bm + jax.lax.broadcasted_iota(
                jnp.int32, (bm, bn), 0)
            cols = pl.program_id(1) * bn + jax.lax.broadcasted_iota(
                jnp.int32, (bm, bn), 1)
            o_ref[...] = jnp.where((rows == cols) & (rows < diag_one_n),
                                   1.0, o_ref[...])


def _mm(a, b, trans_a=False, trans_b=False, bm=256, bn=256, bk=256,
        diag_one_n=None, out_dtype=jnp.float32):
    if trans_a:
        k, m = a.shape
    else:
        m, k = a.shape
    n = b.shape[0] if trans_b else b.shape[1]
    bm = min(bm, m)
    bn = min(bn, n)
    bk = min(bk, k)
    if trans_a:
        a_spec = pl.BlockSpec((bk, bm), lambda i, j, q: (q, i))
    else:
        a_spec = pl.BlockSpec((bm, bk), lambda i, j, q: (i, q))
    if trans_b:
        b_spec = pl.BlockSpec((bn, bk), lambda i, j, q: (j, q))
    else:
        b_spec = pl.BlockSpec((bk, bn), lambda i, j, q: (q, j))
    return pl.pallas_call(
        functools.partial(_mm_kernel, trans_a=trans_a, trans_b=trans_b,
                          diag_one_n=diag_one_n, bm=bm, bn=bn),
        grid=(m // bm, n // bn, k // bk),
        in_specs=[a_spec, b_spec],
        out_specs=pl.BlockSpec((bm, bn), lambda i, j, q: (i, j)),
        out_shape=jax.ShapeDtypeStruct((m, n), out_dtype),
        compiler_params=pltpu.CompilerParams(
            dimension_semantics=("parallel", "parallel", "arbitrary")),
    )(a, b)


def _cast_kernel(a_ref, o_ref):
    o_ref[...] = a_ref[...].astype(o_ref.dtype)


def _to_bf16(a, blk=512):
    m, n = a.shape
    bm, bn = min(blk, m), min(blk, n)
    return pl.pallas_call(
        _cast_kernel,
        grid=(m // bm, n // bn),
        in_specs=[pl.BlockSpec((bm, bn), lambda i, j: (i, j))],
        out_specs=pl.BlockSpec((bm, bn), lambda i, j: (i, j)),
        out_shape=jax.ShapeDtypeStruct((m, n), jnp.bfloat16),
    )(a)


_NW = 32
_EC = 128


def _sc_agg_body(y_hbm, src_hbm, dst_hbm, zrows_hbm, out_hbm,
                 idx_s, idx_d, rows_v, acc, sem):
    c = lax.axis_index("c")
    s = lax.axis_index("s")
    wid = s * 2 + c
    nchunks = src_hbm.shape[1]
    p0 = acc.shape[0]
    stripe = p0 // 16
    pltpu.sync_copy(zrows_hbm, acc.at[pl.ds(s * stripe, stripe)])
    plsc.subcore_barrier()

    def chunk(i, carry):
        pltpu.sync_copy(src_hbm.at[wid, i], idx_s)
        pltpu.async_copy(y_hbm.at[idx_s], rows_v, sem).wait()
        pltpu.sync_copy(dst_hbm.at[wid, i], idx_d)
        pltpu.sync_copy(rows_v, acc.at[idx_d], add=True)
        return carry

    lax.fori_loop(0, nchunks, chunk, 0)
    plsc.subcore_barrier()
    pltpu.sync_copy(acc.at[pl.ds(s * stripe, stripe)],
                    out_hbm.at[c].at[pl.ds(s * stripe, stripe)])


def _sc_edge_agg(y, srcp, dstp, zrows):
    p0 = y.shape[0]
    kern = pl.kernel(
        _sc_agg_body,
        out_type=jax.ShapeDtypeStruct((2, p0, _H), jnp.float32),
        mesh=plsc.VectorSubcoreMesh(core_axis_name="c", subcore_axis_name="s"),
        scratch_types=[
            pltpu.VMEM((_EC,), jnp.int32),
            pltpu.VMEM((_EC,), jnp.int32),
            pltpu.VMEM((_EC, _H), jnp.float32),
            pltpu.VMEM_SHARED((p0, _H), jnp.float32),
            pltpu.SemaphoreType.DMA,
        ],
    )
    return kern(y, srcp, dstp, zrows)


def _head_kernel(g_ref, lw_ref, lb_ref, bg_ref, bb_ref, ow_ref, ob_ref,
                 o_ref):
    inv = 1.0 / jnp.sqrt(1.0 + 1e-5)
    g = g_ref[...]
    for i in range(3):
        g = g * inv * bg_ref[i][None, :] + bb_ref[i][None, :]
        g = jnp.tanh(jnp.dot(g, lw_ref[i], preferred_element_type=jnp.float32)
                     + lb_ref[i][None, :])
    g = g * inv * bg_ref[3][None, :] + bb_ref[3][None, :]
    o_ref[...] = (jnp.dot(g, ow_ref[...], preferred_element_type=jnp.float32)
                  + ob_ref[0][None, :])


def _head(g0, lin_W, lin_b, bn_g, bn_b, out_W_pad, out_b_pad):
    return pl.pallas_call(
        _head_kernel,
        out_shape=jax.ShapeDtypeStruct((_NUM_GRAPHS, _H), jnp.float32),
    )(g0, lin_W, lin_b, bn_g, bn_b, out_W_pad, out_b_pad)


def _pad_rows(x, p):
    return jnp.pad(x, ((0, p - x.shape[0]), (0, 0)))


def _dense_dis(B, mask):
    deg = jnp.sum(B, axis=1) + mask
    return jnp.where(deg > 0.0, deg ** -0.5, 0.0)


def _gcn_dense(B, dis, mask, h_in, W, b, relu):
    y = dis[:, None] * _mm(h_in, W, bm=256, bn=128, bk=128)
    agg = _mm(B, y, bm=256, bn=128, bk=256) + y
    h = dis[:, None] * agg + b[None, :] * mask[:, None]
    return jnp.maximum(h, 0.0) if relu else h


def _score(h, w):
    return jnp.tanh((h @ w) / jnp.linalg.norm(w))


def kernel(x, edge_index, batch, down_W, down_b, pool_w, up_W, up_b,
           lin_W, lin_b, bn_g, bn_b, out_W, out_b):
    f32 = jnp.float32
    n0 = x.shape[0]
    L = [n0]
    for _ in range(_DEPTH):
        L.append(int(math.ceil(0.5 * L[-1])))
    P = [_rup(l) for l in L]

    src = edge_index[0]
    dst = edge_index[1]
    selfloop = src == dst

    selfi = selfloop.astype(jnp.int32)
    cnt = jnp.zeros((2 * n0,), f32).at[dst + n0 * selfi].add(1.0)
    indeg = cnt[:n0] + cnt[n0:]
    selfc = cnt[n0:]
    dfix = jnp.where(selfc == 0.0, 2.0, 0.0)
    dis0 = (indeg + dfix) ** -0.5
    dis0p = jnp.pad(dis0, (0, P[0] - n0))

    ne = edge_index.shape[1]
    npad = _NW * _EC * int(math.ceil(ne / (_NW * _EC)))
    srcp = jnp.pad(src, (0, npad - ne)).reshape(_NW, -1, _EC).astype(jnp.int32)
    dstp = jnp.pad(dst, (0, npad - ne), constant_values=n0)
    dstp = dstp.reshape(_NW, -1, _EC).astype(jnp.int32)
    zrows = jnp.zeros((P[0] // 16, _H), f32)

    def gcn0(h_pad, W, b, relu):
        y = dis0p[:, None] * _mm(h_pad, W, bm=256, bn=128, bk=128)
        parts = _sc_edge_agg(y, srcp, dstp, zrows)
        agg = parts[0, :n0] + parts[1, :n0]
        yl = y[:n0]
        h = dis0[:, None] * (agg + dfix[:, None] * yl) + b[None, :]
        if relu:
            h = jnp.maximum(h, 0.0)
        return _pad_rows(h, P[0])

    x_pad = _pad_rows(x, P[0])
    h0 = gcn0(x_pad, down_W[0], down_b[0], relu=True)

    masks = [(jnp.arange(p) < l).astype(f32) for p, l in zip(P, L)]

    vals1, perm1 = jax.lax.top_k(_score(h0[:n0], pool_w[0]), L[1])
    inv1 = jnp.full((n0,), P[1], jnp.int32).at[perm1].set(
        jnp.arange(L[1], dtype=jnp.int32))
    keep = ~selfloop
    rd = jnp.where(keep, inv1[dst], P[1])
    rs = jnp.where(keep, inv1[src], P[1])
    ar1 = jnp.arange(L[1])
    sz = P[1] * P[0]
    oob = 2 * sz
    fidx = jnp.concatenate([
        jnp.where(rd < P[1], rd * P[0] + src, oob),
        jnp.where(rs < P[1], sz + dst * P[1] + rs, oob),
        ar1 * P[0] + perm1,
        sz + perm1 * P[1] + ar1,
    ])
    flat = jnp.zeros((2 * sz,), f32).at[fidx].add(1.0)
    Ar = flat[:sz].reshape(P[1], P[0])
    Ac = flat[sz:].reshape(P[0], P[1])
    B1 = _mm(_to_bf16(Ar), _to_bf16(Ac), bm=512, bn=512, bk=1024,
             diag_one_n=L[1])
    dis1 = _dense_dis(B1, masks[1])
    h1_in = _pad_rows(h0[:n0][perm1] * vals1[:, None], P[1])
    h1 = _gcn_dense(B1, dis1, masks[1], h1_in, down_W[1], down_b[1], True)

    def next_level(B, h, lvl, pw):
        lp, lc = L[lvl - 1], L[lvl]
        pc = P[lvl]
        vals, perm = jax.lax.top_k(_score(h[:lp], pw), lc)
        permp = jnp.concatenate(
            [perm.astype(jnp.int32), jnp.full((pc - lc,), -1, jnp.int32)])
        hot = permp[:, None] == jnp.arange(B.shape[0], dtype=jnp.int32)[None, :]
        S = hot.astype(f32)
        S16 = hot.astype(jnp.bfloat16)
        B16 = _to_bf16(B)
        Rr = _mm(S16, B16, bm=256, bn=512, bk=256,
                 out_dtype=jnp.bfloat16)
        Rc = _mm(B16, S16, trans_b=True, bm=256, bn=256, bk=512,
                 out_dtype=jnp.bfloat16)
        Bn = _mm(Rr, Rc, bm=256, bn=256, bk=512, diag_one_n=lc)
        h_in = _pad_rows(h[:lp][perm] * vals[:, None], pc)
        return Bn, h_in, S

    B2, h2_in, S2 = next_level(B1, h1, 2, pool_w[1])
    dis2 = _dense_dis(B2, masks[2])
    h2 = _gcn_dense(B2, dis2, masks[2], h2_in, down_W[2], down_b[2], True)

    B3, h3_in, S3 = next_level(B2, h2, 3, pool_w[2])
    dis3 = _dense_dis(B3, masks[3])
    h3 = _gcn_dense(B3, dis3, masks[3], h3_in, down_W[3], down_b[3], True)

    u = _mm(S3, h3, trans_a=True, bm=256, bn=128, bk=256)
    h = _gcn_dense(B2, dis2, masks[2], h2 + u, up_W[0], up_b[0], True)

    u = _mm(S2, h, trans_a=True, bm=256, bn=128, bk=256)
    h = _gcn_dense(B1, dis1, masks[1], h1 + u, up_W[1], up_b[1], True)

    u = jnp.zeros((n0, _H), f32).at[perm1].set(h[:L[1]])
    h = gcn0(_pad_rows(h0[:n0] + u, P[0]), up_W[2], up_b[2], relu=False)

    onehot = (batch[None, :] == jnp.arange(_NUM_GRAPHS)[:, None]).astype(f32)
    onehot = jnp.pad(onehot, ((0, 0), (0, P[0] - n0)))
    g0 = _mm(onehot, h, bm=16, bn=128, bk=256)
    ow = jnp.pad(out_W, ((0, 0), (0, _H - out_W.shape[1])))
    ob = jnp.pad(out_b, (0, _H - out_b.shape[0]))[None, :]
    out = _head(g0, lin_W, lin_b, bn_g, bn_b, ow, ob)
    return out[:, :out_W.shape[1]]
